# baseline (speedup 1.0000x reference)
import functools

import jax
import jax.numpy as jnp
from jax import lax
from jax.experimental import pallas as pl
from jax.experimental.pallas import tpu as pltpu
from jax.experimental.pallas import tpu_sc as plsc

_B = 16384
_D = 32
_L = 16
_NC = 2
_NS = 16
_NW = _NC * _NS
_BPW = _B // _NW

_TGRID = -(-1000000 // 128)
_OCT = _TGRID * 1024

_mesh = plsc.VectorSubcoreMesh(core_axis_name="c", subcore_axis_name="s")


def _body(ub_hbm, ib_hbm, ut_hbm, it_hbm, out_hbm,
          uidx_v, iidx_v, uoff, ioff, ubuf, ibuf, out_v, sem_u, sem_i):
    wid = lax.axis_index("s") * _NC + lax.axis_index("c")
    base = wid * _BPW

    pltpu.sync_copy(ub_hbm.at[pl.ds(base, _BPW)], uidx_v)
    pltpu.sync_copy(ib_hbm.at[pl.ds(base, _BPW)], iidx_v)

    def offsets(v, carry):
        s = v * _L
        ru = uidx_v[pl.ds(s, _L)]
        ri = iidx_v[pl.ds(s, _L)]
        bu = ru
        bi = ri
        for c in range(_D):
            k = c * 1000000
            uoff[pl.ds(c * _BPW + s, _L)] = bu + k
            ioff[pl.ds(c * _BPW + s, _L)] = bi + k
        return carry

    lax.fori_loop(0, _BPW // _L, offsets, 0)

    cu = pltpu.async_copy(ut_hbm.at[0].at[uoff], ubuf, sem_u)
    ci = pltpu.async_copy(it_hbm.at[0].at[ioff], ibuf, sem_i)
    cu.wait()
    ci.wait()

    def group(g, carry):
        b = g * _L
        acc = jnp.zeros((_L,), jnp.float32)
        for c in range(_D):
            acc = (acc + ubuf[pl.ds(c * _BPW + b, _L)]
                   * ibuf[pl.ds(c * _BPW + b, _L)])
        out_v[pl.ds(b, _L)] = acc
        return carry

    lax.fori_loop(0, _BPW // _L, group, 0)

    pltpu.sync_copy(out_v, out_hbm.at[pl.ds(base, _BPW)])


@jax.jit
def _run(user_batch, item_batch, user_table_t, item_table_t):
    k = functools.partial(
        pl.kernel,
        out_type=jax.ShapeDtypeStruct((_B,), jnp.float32),
        mesh=_mesh,
        scratch_types=[
            pltpu.VMEM((_BPW,), jnp.int32),
            pltpu.VMEM((_BPW,), jnp.int32),
            pltpu.VMEM((_D * _BPW,), jnp.int32),
            pltpu.VMEM((_D * _BPW,), jnp.int32),
            pltpu.VMEM((_D * _BPW,), jnp.float32),
            pltpu.VMEM((_D * _BPW,), jnp.float32),
            pltpu.VMEM((_BPW,), jnp.float32),
            pltpu.SemaphoreType.DMA,
            pltpu.SemaphoreType.DMA,
        ],
        compiler_params=pltpu.CompilerParams(
            needs_layout_passes=False, use_tc_tiling_on_sc=False),
    )(_body)
    return k(user_batch, item_batch, user_table_t, item_table_t)


def kernel(user_batch, item_batch, user_table, item_table):
    return _run(user_batch.astype(jnp.int32), item_batch.astype(jnp.int32),
                user_table.T, item_table.T)
